# Initial kernel scaffold; baseline (speedup 1.0000x reference)
import jax
import jax.numpy as jnp
from jax.experimental import pallas as pl


def kernel(x, W, b):
    raise NotImplementedError("write your pallas kernel here")


# speedup vs baseline: 3.0570x; 3.0570x over previous
import jax
import jax.numpy as jnp
from jax.experimental import pallas as pl

K = 8


def _router_body(x_ref, w_ref, b_ref, ow_ref, oi_ref):
    xb = x_ref[0]
    logits = jax.lax.dot_general(
        xb, w_ref[...], (((0,), (1,)), ((), ())),
        preferred_element_type=jnp.float32,
    ) + b_ref[...]
    T, E = logits.shape
    col = jax.lax.broadcasted_iota(jnp.int32, (T, E), 1)
    neg_inf = jnp.float32(-jnp.inf)
    vals, idxs = [], []
    cur = logits
    for _ in range(K):
        m = jnp.max(cur, axis=1, keepdims=True)
        idx = jnp.min(jnp.where(cur == m, col, E), axis=1, keepdims=True)
        vals.append(m)
        idxs.append(idx)
        cur = jnp.where(col == idx, neg_inf, cur)
    top_vals = jnp.concatenate(vals, axis=1)
    top_idx = jnp.concatenate(idxs, axis=1)
    e = jnp.exp(top_vals - top_vals[:, 0:1])
    w = e / jnp.sum(e, axis=1, keepdims=True)
    ow_ref[0] = w
    oi_ref[0] = top_idx


def kernel(x, W, b):
    B, C, H, Wd = x.shape
    T = H * Wd
    E = W.shape[0]
    xr = x.reshape(B, C, T)
    b2 = b.reshape(1, E)
    wout, iout = pl.pallas_call(
        _router_body,
        grid=(B,),
        in_specs=[
            pl.BlockSpec((1, C, T), lambda i: (i, 0, 0)),
            pl.BlockSpec((E, C), lambda i: (0, 0)),
            pl.BlockSpec((1, E), lambda i: (0, 0)),
        ],
        out_specs=[
            pl.BlockSpec((1, T, K), lambda i: (i, 0, 0)),
            pl.BlockSpec((1, T, K), lambda i: (i, 0, 0)),
        ],
        out_shape=[
            jax.ShapeDtypeStruct((B, T, K), jnp.float32),
            jax.ShapeDtypeStruct((B, T, K), jnp.int32),
        ],
    )(xr, W, b2)
    return wout.reshape(B, H, Wd, K), iout.reshape(B, H, Wd, K)
